# baseline (speedup 1.0000x reference)
import jax
import jax.numpy as jnp
from jax.experimental import pallas as pl
from jax.experimental.pallas import tpu as pltpu
from jax.experimental.pallas import tpu_sc as plsc

Q = 16384
D = 768
K = 10000
KPAD = 10240
BQ = 512
BK = 2048
NQ = Q // BQ
NK = KPAD // BK
GW = 128
TW = 128


def _nearest_body(f_ref, ct_ref, out_ref, bv_ref, bi_ref, csq_ref,
                  fn_ref, fsq_ref):
    q = pl.program_id(0)
    k = pl.program_id(1)

    @pl.when(q == 0)
    def _():
        c = ct_ref[...]
        csq_ref[0, pl.ds(k * BK, BK)] = jnp.sum(c * c, axis=0)

    @pl.when(k == 0)
    def _():
        f = f_ref[...]
        nrm = jnp.sqrt(jnp.sum(f * f, axis=1, keepdims=True))
        fn = f / jnp.maximum(nrm, 1e-12)
        fn_ref[...] = fn
        fsq_ref[...] = jnp.sum(fn * fn, axis=1, keepdims=True)
        bv_ref[...] = jnp.full((BQ, 1), jnp.inf, jnp.float32)
        bi_ref[...] = jnp.zeros((BQ, 1), jnp.int32)

    dot = jax.lax.dot_general(fn_ref[...], ct_ref[...],
                              (((1,), (0,)), ((), ())),
                              preferred_element_type=jnp.float32,
                              precision=jax.lax.Precision.HIGHEST)
    d2 = fsq_ref[...] + csq_ref[0, pl.ds(k * BK, BK)][None, :] - 2.0 * dot
    d = jnp.sqrt(jnp.maximum(d2, 0.0))
    col = k * BK + jax.lax.broadcasted_iota(jnp.int32, (BQ, BK), 1)
    d = jnp.where(col < K, d, jnp.inf)

    cmin = jnp.min(d, axis=1, keepdims=True)
    cidx = jnp.min(jnp.where(d == cmin, col, jnp.int32(2**31 - 1)),
                   axis=1, keepdims=True)

    bv = bv_ref[...]
    take = cmin < bv
    bv_ref[...] = jnp.where(take, cmin, bv)
    bi_ref[...] = jnp.where(take, cidx, bi_ref[...])

    @pl.when(k == NK - 1)
    def _():
        out_ref[...] = bi_ref[...][None]


def _nearest(features, centroids):
    ct = jnp.pad(centroids, ((0, KPAD - K), (0, 0))).T
    out = pl.pallas_call(
        _nearest_body,
        grid=(NQ, NK),
        in_specs=[
            pl.BlockSpec((BQ, D), lambda q, k: (q, 0)),
            pl.BlockSpec((D, BK), lambda q, k: (0, k)),
        ],
        out_specs=pl.BlockSpec((1, BQ, 1), lambda q, k: (q, 0, 0)),
        out_shape=jax.ShapeDtypeStruct((NQ, BQ, 1), jnp.int32),
        scratch_shapes=[
            pltpu.VMEM((BQ, 1), jnp.float32),
            pltpu.VMEM((BQ, 1), jnp.int32),
            pltpu.VMEM((1, KPAD), jnp.float32),
            pltpu.VMEM((BQ, D), jnp.float32),
            pltpu.VMEM((BQ, 1), jnp.float32),
        ],
    )(features, ct)
    return out.reshape(Q)


def _gather_sc(task_ids, nearest):
    t2 = jnp.broadcast_to(task_ids.reshape(K, 1), (K, TW))
    idx = nearest.reshape(1, Q)
    mesh = plsc.VectorSubcoreMesh(core_axis_name="core",
                                  subcore_axis_name="subcore")

    @pl.kernel(out_type=jax.ShapeDtypeStruct((Q, TW), task_ids.dtype),
               mesh=mesh)
    def knl(t_hbm, i_hbm, o_hbm):
        def body(i_vmem, o_vmem):
            pltpu.sync_copy(t_hbm.at[i_vmem.at[0]], o_vmem)

        pltpu.emit_pipeline(
            body,
            grid=(Q // GW,),
            in_specs=[pl.BlockSpec((1, GW), index_map=lambda i: (0, i))],
            out_specs=[pl.BlockSpec((GW, TW), index_map=lambda i: (i, 0))],
            core_axis_name="subcore",
            dimension_semantics=(pltpu.PARALLEL,),
        )(i_hbm, o_hbm)

    return knl(t2, idx)[:, 0].reshape(Q)


def kernel(features, centroids, task_ids):
    nearest = _nearest(features, centroids)
    return _gather_sc(task_ids, nearest)
